# baseline (speedup 1.0000x reference)
import functools

import jax
import jax.numpy as jnp
from jax import lax
from jax.experimental import pallas as pl
from jax.experimental.pallas import tpu as pltpu
from jax.experimental.pallas import tpu_sc as plsc

NUM_USERS = 1000000
NUM_ITEMS = 1000000
NUM_LABELS = 5
EMBED_DIM = 32
BATCH = 16384

_BETA_PACK = 4
_BETA_W = _BETA_PACK * (NUM_LABELS - 1)

_INFO = plsc.get_sparse_core_info()
_NC, _NS, _LANES = _INFO.num_cores, _INFO.num_subcores, _INFO.num_lanes
_NW = _NC * _NS
_BPW = BATCH // _NW
_IDX_CHUNK = 128
_NCHUNK = _BPW // _IDX_CHUNK
_NGROUP = _BPW // _LANES

_mesh = plsc.VectorSubcoreMesh(core_axis_name="c", subcore_axis_name="s")


@functools.partial(
    pl.kernel,
    mesh=_mesh,
    compiler_params=pltpu.CompilerParams(
        needs_layout_passes=False, use_tc_tiling_on_sc=False),
    out_type=jax.ShapeDtypeStruct((BATCH, NUM_LABELS), jnp.float32),
    scratch_types=[
        pltpu.VMEM((_BPW,), jnp.int32),
        pltpu.VMEM((_NCHUNK, _IDX_CHUNK), jnp.int32),
        pltpu.VMEM((_NCHUNK, _IDX_CHUNK), jnp.int32),
        pltpu.VMEM((_BPW, EMBED_DIM), jnp.float32),
        pltpu.VMEM((_BPW, EMBED_DIM), jnp.float32),
        pltpu.VMEM((_BPW, _BETA_W), jnp.float32),
        pltpu.VMEM((_BPW, NUM_LABELS), jnp.float32),
        pltpu.SemaphoreType.DMA,
    ],
)
def _ordrec_sc(uid_hbm, iid_hbm, uemb_hbm, iemb_hbm, ubeta_hbm, out_hbm,
               uid_v, iid_v, bid_v, u_rows, i_rows, b_rows, out_v, sem):
    wid = lax.axis_index("s") * _NC + lax.axis_index("c")
    base = wid * _BPW

    pltpu.sync_copy(uid_hbm.at[pl.ds(base, _BPW)], uid_v)
    for j in range(_NCHUNK):
        pltpu.sync_copy(iid_hbm.at[pl.ds(base + j * _IDX_CHUNK, _IDX_CHUNK)],
                        iid_v.at[j])
    for j in range(_NCHUNK):
        for k in range(_IDX_CHUNK // _LANES):
            sl = pl.ds(j * _IDX_CHUNK + k * _LANES, _LANES)
            bid_v[j, pl.ds(k * _LANES, _LANES)] = (
                lax.shift_right_logical(uid_v[sl], 2))

    copies = []
    for j in range(_NCHUNK):
        sl = pl.ds(j * _IDX_CHUNK, _IDX_CHUNK)
        copies.append(pltpu.async_copy(uemb_hbm.at[uid_v.at[sl]],
                                       u_rows.at[sl], sem))
        copies.append(pltpu.async_copy(iemb_hbm.at[iid_v.at[j]],
                                       i_rows.at[sl], sem))
        copies.append(pltpu.async_copy(ubeta_hbm.at[bid_v.at[j]],
                                       b_rows.at[sl], sem))
    for c in copies:
        c.wait()

    lanes = lax.iota(jnp.int32, _LANES)

    def group(g, carry):
        rows = g * _LANES + lanes
        y = jnp.zeros((_LANES,), jnp.float32)
        for d in range(EMBED_DIM):
            col = jnp.full((_LANES,), d, jnp.int32)
            y = y + (plsc.load_gather(u_rows, [rows, col]) *
                     plsc.load_gather(i_rows, [rows, col]))
        uidv = plsc.load_gather(uid_v, [rows])
        boff = lax.shift_left((uidv & 3), 2)
        c0 = plsc.load_gather(b_rows, [rows, boff])
        s_prev = 1.0 / (1.0 + jnp.exp(y - c0))
        plsc.store_scatter(out_v, [rows, jnp.zeros((_LANES,), jnp.int32)],
                           s_prev)
        cum = c0
        for j in range(1, NUM_LABELS - 1):
            cum = cum + jnp.exp(plsc.load_gather(b_rows, [rows, boff + j]))
            s = 1.0 / (1.0 + jnp.exp(y - cum))
            plsc.store_scatter(out_v,
                               [rows, jnp.full((_LANES,), j, jnp.int32)],
                               s - s_prev)
            s_prev = s
        plsc.store_scatter(
            out_v, [rows, jnp.full((_LANES,), NUM_LABELS - 1, jnp.int32)],
            1.0 - s_prev)
        return carry

    lax.fori_loop(0, _NGROUP, group, jnp.int32(0))

    pltpu.sync_copy(out_v, out_hbm.at[pl.ds(base, _BPW)])


def kernel(user_ids, item_ids, user_embeddings, item_embeddings, user_betas):
    packed_betas = user_betas.reshape(NUM_USERS // _BETA_PACK, _BETA_W)
    return _ordrec_sc(user_ids, item_ids, user_embeddings, item_embeddings,
                      packed_betas)
